# baseline (speedup 1.0000x reference)
import jax
import jax.numpy as jnp
from jax.experimental import pallas as pl
from jax.experimental.pallas import tpu as pltpu

_BI = 1024


def _sage_block(adj_ref, h_ref, wt_ref, out_ref, hw16_ref):
    i = pl.program_id(0)

    @pl.when(i == 0)
    def _precompute_hw():
        hw = jnp.dot(h_ref[...], wt_ref[...],
                     preferred_element_type=jnp.float32)
        hw16_ref[...] = hw.astype(jnp.bfloat16)

    a16 = jnp.minimum(adj_ref[...], 1.0).astype(jnp.bfloat16)
    out_ref[...] = jnp.dot(a16, hw16_ref[...],
                           preferred_element_type=jnp.float32)


def kernel(h, adj, W):
    n, d_in = h.shape
    d_out = W.shape[0]
    wt = W.T
    grid = (n // _BI,)
    return pl.pallas_call(
        _sage_block,
        grid=grid,
        in_specs=[
            pl.BlockSpec((_BI, n), lambda i: (i, 0)),
            pl.BlockSpec((n, d_in), lambda i: (0, 0)),
            pl.BlockSpec((d_in, d_out), lambda i: (0, 0)),
        ],
        out_specs=pl.BlockSpec((_BI, d_out), lambda i: (i, 0)),
        out_shape=jax.ShapeDtypeStruct((n, d_out), jnp.float32),
        scratch_shapes=[
            pltpu.VMEM((n, d_out), jnp.bfloat16),
        ],
        compiler_params=pltpu.CompilerParams(
            dimension_semantics=("arbitrary",),
        ),
    )(adj, h, wt)
